# Initial kernel scaffold; baseline (speedup 1.0000x reference)
import jax
import jax.numpy as jnp
from jax.experimental import pallas as pl


def kernel(input_ids, offsets, edge_index, self_w, ppi_w, emb_table, input_bias, bn0_g, bn0_b, W1, b1, bn1_g, bn1_b, W2, b2, bn2_g, bn2_b):
    raise NotImplementedError("write your pallas kernel here")


# speedup vs baseline: 2.7124x; 2.7124x over previous
import functools
import math

import jax
import jax.numpy as jnp
from jax.experimental import pallas as pl
from jax.experimental.pallas import tpu as pltpu

N_NODES = 10000
HIDDEN = 256
BAG = 10
_BN_S = 1.0 / math.sqrt(1.0 + 1e-5)


def _h0_body(emb_ref, sg_ref, bb_ref, out_ref):
    out_ref[...] = jax.nn.relu(emb_ref[...] * sg_ref[...] + bb_ref[...])


def _h0_stage(emb, input_bias, g, b):
    sg = (g * _BN_S)[None, :]
    bb = (b + input_bias * _BN_S * g)[None, :]
    return pl.pallas_call(
        _h0_body,
        out_shape=jax.ShapeDtypeStruct((N_NODES, HIDDEN), jnp.float32),
        grid=(10,),
        in_specs=[
            pl.BlockSpec((N_NODES // 10, HIDDEN), lambda i: (i, 0)),
            pl.BlockSpec((1, HIDDEN), lambda i: (0, 0)),
            pl.BlockSpec((1, HIDDEN), lambda i: (0, 0)),
        ],
        out_specs=pl.BlockSpec((N_NODES // 10, HIDDEN), lambda i: (i, 0)),
    )(emb, sg, bb)


def _layer_body(ppi_ref, res_ref, wt_ref, sg_ref, bb_ref, out_ref):
    acc = jnp.dot(ppi_ref[...], wt_ref[...], preferred_element_type=jnp.float32)
    out_ref[...] = jax.nn.relu(acc * sg_ref[...] + bb_ref[...]) + res_ref[...]


def _layer_stage(ppi, res, W, b, g, bb):
    sg = (g * _BN_S)[None, :]
    bb2 = (bb + b * _BN_S * g)[None, :]
    wt = W.T
    return pl.pallas_call(
        _layer_body,
        out_shape=jax.ShapeDtypeStruct((N_NODES, HIDDEN), jnp.float32),
        grid=(10,),
        in_specs=[
            pl.BlockSpec((N_NODES // 10, HIDDEN), lambda i: (i, 0)),
            pl.BlockSpec((N_NODES // 10, HIDDEN), lambda i: (i, 0)),
            pl.BlockSpec((HIDDEN, HIDDEN), lambda i: (0, 0)),
            pl.BlockSpec((1, HIDDEN), lambda i: (0, 0)),
            pl.BlockSpec((1, HIDDEN), lambda i: (0, 0)),
        ],
        out_specs=pl.BlockSpec((N_NODES // 10, HIDDEN), lambda i: (i, 0)),
    )(ppi, res, wt, sg, bb2)


def kernel(input_ids, offsets, edge_index, self_w, ppi_w, emb_table, input_bias,
           bn0_g, bn0_b, W1, b1, bn1_g, bn1_b, W2, b2, bn2_g, bn2_b):
    rows = jnp.take(emb_table, input_ids, axis=0)
    emb = rows.reshape(N_NODES, BAG, HIDDEN).sum(axis=1)
    h = _h0_stage(emb, input_bias, bn0_g, bn0_b)
    src = edge_index[0]
    dst = edge_index[1]
    for (W, b, g, bb) in ((W1, b1, bn1_g, bn1_b), (W2, b2, bn2_g, bn2_b)):
        hs = h[src]
        res = jax.ops.segment_sum(hs * self_w[:, None], dst, num_segments=N_NODES)
        ppi = jax.ops.segment_sum(hs * ppi_w[:, None], dst, num_segments=N_NODES)
        h = _layer_stage(ppi, res, W, b, g, bb)
    return h


# speedup vs baseline: 5.4946x; 2.0257x over previous
import functools
import math

import jax
import jax.numpy as jnp
from jax import lax
from jax.experimental import pallas as pl
from jax.experimental.pallas import tpu as pltpu
from jax.experimental.pallas import tpu_sc as plsc

N_NODES = 10000
HIDDEN = 256
BAG = 10
N_EDGES = 160000
_BN_S = 1.0 / math.sqrt(1.0 + 1e-5)

NC = 2
NS = 16
NW = NC * NS
Q = 4
FQ = HIDDEN // Q

BPW = 320
CB = 16
NCHUNK_B = BPW // CB


def _emb_body(ids_hbm, table_hbm, sg_hbm, bb_hbm, out_hbm,
              idx_a, idx_b, rows_v, sg_v, bb_v, oq_v, sem):
    c = lax.axis_index("c")
    s = lax.axis_index("s")
    w = c * NS + s
    pltpu.sync_copy(sg_hbm, sg_v)
    pltpu.sync_copy(bb_hbm, bb_v)

    def chunk(i, carry):
        start = jnp.minimum(w * BPW + i * CB, N_NODES - CB)
        half = CB * BAG // 2
        pltpu.sync_copy(ids_hbm.at[pl.ds(start * BAG, half)], idx_a)
        pltpu.sync_copy(ids_hbm.at[pl.ds(start * BAG + half, half)], idx_b)
        cp_a = pltpu.async_copy(table_hbm.at[idx_a], rows_v.at[pl.ds(0, half)], sem)
        cp_b = pltpu.async_copy(table_hbm.at[idx_b], rows_v.at[pl.ds(half, half)], sem)
        cp_a.wait()
        cp_b.wait()

        def bag(b, carry2):
            base = b * BAG
            for f in range(HIDDEN // 16):
                acc = rows_v[base, pl.ds(f * 16, 16)]
                for k in range(1, BAG):
                    acc = acc + rows_v[base + k, pl.ds(f * 16, 16)]
                val = acc * sg_v[pl.ds(f * 16, 16)] + bb_v[pl.ds(f * 16, 16)]
                val = jnp.maximum(val, 0.0)
                q, fq = f // (FQ // 16), f % (FQ // 16)
                oq_v[q, b, pl.ds(fq * 16, 16)] = val
            return carry2

        lax.fori_loop(0, CB, bag, 0)
        for q in range(Q):
            pltpu.sync_copy(oq_v.at[q], out_hbm.at[pl.ds(q * N_NODES + start, CB)])
        return carry

    lax.fori_loop(0, NCHUNK_B, chunk, 0)


def _emb_stage(input_ids, emb_table, sg, bb):
    mesh = plsc.VectorSubcoreMesh(core_axis_name="c", subcore_axis_name="s")
    return pl.kernel(
        _emb_body,
        out_type=jax.ShapeDtypeStruct((Q * N_NODES, FQ), jnp.float32),
        mesh=mesh,
        scratch_types=[
            pltpu.VMEM((CB * BAG // 2,), jnp.int32),
            pltpu.VMEM((CB * BAG // 2,), jnp.int32),
            pltpu.VMEM((CB * BAG, HIDDEN), jnp.float32),
            pltpu.VMEM((HIDDEN,), jnp.float32),
            pltpu.VMEM((HIDDEN,), jnp.float32),
            pltpu.VMEM((Q, CB, FQ), jnp.float32),
            pltpu.SemaphoreType.DMA,
        ],
    )(input_ids, emb_table, sg, bb)


CE = 80
EPT = N_EDGES // NS
NCHUNK_E = EPT // CE
FLUSH_R = 640


def _edge_body(h_hbm, src_hbm, dst_hbm, sw_hbm, pw_hbm, z_hbm,
               res_hbm, ppi_hbm,
               idx_v, src_v, dst_v, sw_v, pw_v, rows_v, sres_v, sppi_v,
               res_acc, ppi_acc, sem):
    c = lax.axis_index("c")
    s = lax.axis_index("s")
    ebase = s * EPT
    for p in range(2):
        q = c * 2 + p
        qbase = q * N_NODES
        fb = jnp.minimum(s * FLUSH_R, N_NODES - FLUSH_R)
        pltpu.sync_copy(z_hbm, res_acc.at[pl.ds(fb, FLUSH_R)])
        pltpu.sync_copy(z_hbm, ppi_acc.at[pl.ds(fb, FLUSH_R)])
        plsc.subcore_barrier()

        def chunk(i, carry):
            eb = ebase + i * CE
            pltpu.sync_copy(src_hbm.at[pl.ds(eb, CE)], src_v)
            pltpu.sync_copy(dst_hbm.at[pl.ds(eb, CE)], dst_v)
            pltpu.sync_copy(sw_hbm.at[pl.ds(eb, CE)], sw_v)
            pltpu.sync_copy(pw_hbm.at[pl.ds(eb, CE)], pw_v)
            for t in range(CE // 16):
                idx_v[pl.ds(t * 16, 16)] = src_v[pl.ds(t * 16, 16)] + qbase
            pltpu.async_copy(h_hbm.at[idx_v], rows_v, sem).wait()

            def group(g, carry2):
                ws16 = sw_v[pl.ds(g * 16, 16)]
                wp16 = pw_v[pl.ds(g * 16, 16)]
                for j2 in range(16):
                    j = g * 16 + j2
                    wsv = jnp.full((16,), ws16[j2], dtype=jnp.float32)
                    wpv = jnp.full((16,), wp16[j2], dtype=jnp.float32)
                    for f in range(FQ // 16):
                        r = rows_v[j, pl.ds(f * 16, 16)]
                        sres_v[j, pl.ds(f * 16, 16)] = r * wsv
                        sppi_v[j, pl.ds(f * 16, 16)] = r * wpv
                return carry2

            lax.fori_loop(0, CE // 16, group, 0)
            pltpu.sync_copy(sres_v, res_acc.at[dst_v], add=True)
            pltpu.sync_copy(sppi_v, ppi_acc.at[dst_v], add=True)
            return carry

        lax.fori_loop(0, NCHUNK_E, chunk, 0)
        plsc.subcore_barrier()
        pltpu.sync_copy(res_acc.at[pl.ds(fb, FLUSH_R)],
                        res_hbm.at[pl.ds(qbase + fb, FLUSH_R)])
        pltpu.sync_copy(ppi_acc.at[pl.ds(fb, FLUSH_R)],
                        ppi_hbm.at[pl.ds(qbase + fb, FLUSH_R)])
        plsc.subcore_barrier()


def _edge_stage(hflat, src, dst, self_w, ppi_w, zrows):
    mesh = plsc.VectorSubcoreMesh(core_axis_name="c", subcore_axis_name="s")
    return pl.kernel(
        _edge_body,
        out_type=(jax.ShapeDtypeStruct((Q * N_NODES, FQ), jnp.float32),
                  jax.ShapeDtypeStruct((Q * N_NODES, FQ), jnp.float32)),
        mesh=mesh,
        compiler_params=pltpu.CompilerParams(use_tc_tiling_on_sc=False),
        scratch_types=[
            pltpu.VMEM((CE,), jnp.int32),
            pltpu.VMEM((CE,), jnp.int32),
            pltpu.VMEM((CE,), jnp.int32),
            pltpu.VMEM((CE,), jnp.float32),
            pltpu.VMEM((CE,), jnp.float32),
            pltpu.VMEM((CE, FQ), jnp.float32),
            pltpu.VMEM((CE, FQ), jnp.float32),
            pltpu.VMEM((CE, FQ), jnp.float32),
            pltpu.VMEM_SHARED((N_NODES, FQ), jnp.float32),
            pltpu.VMEM_SHARED((N_NODES, FQ), jnp.float32),
            pltpu.SemaphoreType.DMA,
        ],
    )(hflat, src, dst, self_w, ppi_w, zrows)


_RB = 1000


def _layer_body_split(ppi_ref, res_ref, wt_ref, sg_ref, bb_ref, out_ref):
    acc = jnp.dot(ppi_ref[0], wt_ref[0], preferred_element_type=jnp.float32)
    for q in range(1, Q):
        acc = acc + jnp.dot(ppi_ref[q], wt_ref[q], preferred_element_type=jnp.float32)
    z = jnp.maximum(acc * sg_ref[...] + bb_ref[...], 0.0)
    for q in range(Q):
        out_ref[q] = z[:, q * FQ:(q + 1) * FQ] + res_ref[q]


def _layer_body_final(ppi_ref, res_ref, wt_ref, sg_ref, bb_ref, out_ref):
    acc = jnp.dot(ppi_ref[0], wt_ref[0], preferred_element_type=jnp.float32)
    for q in range(1, Q):
        acc = acc + jnp.dot(ppi_ref[q], wt_ref[q], preferred_element_type=jnp.float32)
    z = jnp.maximum(acc * sg_ref[...] + bb_ref[...], 0.0)
    cat = jnp.concatenate([res_ref[q] for q in range(Q)], axis=1)
    out_ref[...] = z + cat


def _layer_stage(ppi4, res4, W, b, g, bb, final):
    sg = (g * _BN_S)[None, :]
    bb2 = (bb + b * _BN_S * g)[None, :]
    wt4 = W.T.reshape(Q, FQ, HIDDEN)
    grid = (N_NODES // _RB,)
    split_spec = pl.BlockSpec((Q, _RB, FQ), lambda i: (0, i, 0))
    if final:
        body, out_shape = _layer_body_final, jax.ShapeDtypeStruct((N_NODES, HIDDEN), jnp.float32)
        out_spec = pl.BlockSpec((_RB, HIDDEN), lambda i: (i, 0))
    else:
        body, out_shape = _layer_body_split, jax.ShapeDtypeStruct((Q, N_NODES, FQ), jnp.float32)
        out_spec = split_spec
    return pl.pallas_call(
        body,
        out_shape=out_shape,
        grid=grid,
        in_specs=[
            split_spec,
            split_spec,
            pl.BlockSpec((Q, FQ, HIDDEN), lambda i: (0, 0, 0)),
            pl.BlockSpec((1, HIDDEN), lambda i: (0, 0)),
            pl.BlockSpec((1, HIDDEN), lambda i: (0, 0)),
        ],
        out_specs=out_spec,
    )(ppi4, res4, wt4, sg, bb2)


def kernel(input_ids, offsets, edge_index, self_w, ppi_w, emb_table, input_bias,
           bn0_g, bn0_b, W1, b1, bn1_g, bn1_b, W2, b2, bn2_g, bn2_b):
    sg0 = bn0_g * _BN_S
    bb0 = bn0_b + input_bias * _BN_S * bn0_g
    hflat = _emb_stage(input_ids, emb_table, sg0, bb0)
    src = edge_index[0]
    dst = edge_index[1]
    zrows = jnp.zeros((FLUSH_R, FQ), jnp.float32)
    for (W, b, g, bb, final) in ((W1, b1, bn1_g, bn1_b, False),
                                 (W2, b2, bn2_g, bn2_b, True)):
        resflat, ppiflat = _edge_stage(hflat, src, dst, self_w, ppi_w, zrows)
        out = _layer_stage(ppiflat.reshape(Q, N_NODES, FQ),
                           resflat.reshape(Q, N_NODES, FQ), W, b, g, bb, final)
        if not final:
            hflat = out.reshape(Q * N_NODES, FQ)
    return out


# speedup vs baseline: 8.7103x; 1.5853x over previous
import functools
import math

import jax
import jax.numpy as jnp
from jax import lax
from jax.experimental import pallas as pl
from jax.experimental.pallas import tpu as pltpu
from jax.experimental.pallas import tpu_sc as plsc

N_NODES = 10000
HIDDEN = 256
BAG = 10
N_EDGES = 160000
_BN_S = 1.0 / math.sqrt(1.0 + 1e-5)

NC = 2
NS = 16
NW = NC * NS
Q = 4
FQ = HIDDEN // Q

BPW = 320
CB = 16
NCHUNK_B = BPW // CB


def _emb_body(ids_hbm, table_hbm, sg_hbm, bb_hbm, out_hbm,
              idx_a, idx_b, rows_v, sg_v, bb_v, oq_v, sem):
    c = lax.axis_index("c")
    s = lax.axis_index("s")
    w = c * NS + s
    pltpu.sync_copy(sg_hbm, sg_v)
    pltpu.sync_copy(bb_hbm, bb_v)

    def chunk(i, carry):
        start = jnp.minimum(w * BPW + i * CB, N_NODES - CB)
        half = CB * BAG // 2
        pltpu.sync_copy(ids_hbm.at[pl.ds(start * BAG, half)], idx_a)
        pltpu.sync_copy(ids_hbm.at[pl.ds(start * BAG + half, half)], idx_b)
        cp_a = pltpu.async_copy(table_hbm.at[idx_a], rows_v.at[pl.ds(0, half)], sem)
        cp_b = pltpu.async_copy(table_hbm.at[idx_b], rows_v.at[pl.ds(half, half)], sem)
        cp_a.wait()
        cp_b.wait()

        def bag(b, carry2):
            base = b * BAG
            for f in range(HIDDEN // 16):
                acc = rows_v[base, pl.ds(f * 16, 16)]
                for k in range(1, BAG):
                    acc = acc + rows_v[base + k, pl.ds(f * 16, 16)]
                val = acc * sg_v[pl.ds(f * 16, 16)] + bb_v[pl.ds(f * 16, 16)]
                val = jnp.maximum(val, 0.0)
                q, fq = f // (FQ // 16), f % (FQ // 16)
                oq_v[q, b, pl.ds(fq * 16, 16)] = val
            return carry2

        lax.fori_loop(0, CB, bag, 0)
        for q in range(Q):
            pltpu.sync_copy(oq_v.at[q], out_hbm.at[pl.ds(q * N_NODES + start, CB)])
        return carry

    lax.fori_loop(0, NCHUNK_B, chunk, 0)


def _emb_stage(input_ids, emb_table, sg, bb):
    mesh = plsc.VectorSubcoreMesh(core_axis_name="c", subcore_axis_name="s")
    return pl.kernel(
        _emb_body,
        out_type=jax.ShapeDtypeStruct((Q * N_NODES, FQ), jnp.float32),
        mesh=mesh,
        scratch_types=[
            pltpu.VMEM((CB * BAG // 2,), jnp.int32),
            pltpu.VMEM((CB * BAG // 2,), jnp.int32),
            pltpu.VMEM((CB * BAG, HIDDEN), jnp.float32),
            pltpu.VMEM((HIDDEN,), jnp.float32),
            pltpu.VMEM((HIDDEN,), jnp.float32),
            pltpu.VMEM((Q, CB, FQ), jnp.float32),
            pltpu.SemaphoreType.DMA,
        ],
    )(input_ids, emb_table, sg, bb)


CE = 64
NCHUNK_E = 160
EPT_PAD = NCHUNK_E * CE
FLUSH_R = 640


def _edge_body(h_hbm, meta_hbm, wmeta_hbm, z_hbm, rp_hbm,
               meta_v, sidx_v, didx_v, wbuf_v, rows_v, stg_v, rp_acc,
               sem_g0, sem_g1, sem_g2, sem_g3,
               sem_w0, sem_w1, sem_w2, sem_w3,
               sem_s0a, sem_s0b, sem_s1a, sem_s1b):
    c = lax.axis_index("c")
    s = lax.axis_index("s")
    pltpu.sync_copy(meta_hbm.at[s], meta_v)
    dummy_g = h_hbm.at[pl.ds(0, CE)]
    dummy_s = rp_hbm.at[pl.ds(0, CE // 2)]
    dummy_w = wmeta_hbm.at[0, 0]
    sem_g = (sem_g0, sem_g1, sem_g2, sem_g3)
    sem_w = (sem_w0, sem_w1, sem_w2, sem_w3)
    sem_s = ((sem_s0a, sem_s0b), (sem_s1a, sem_s1b))

    def unpack(i, b4, qbase):
        for t in range(CE // 16):
            m = meta_v[i, pl.ds(t * 16, 16)]
            sidx_v[b4, pl.ds(t * 16, 16)] = (m & 0xFFFF) + qbase
            didx_v[b4, t // 2, pl.ds((t % 2) * 16, 16)] = lax.shift_right_logical(m, 16)

    for p in range(2):
        q = c * 2 + p
        qbase = q * N_NODES
        fb = jnp.minimum(s * FLUSH_R, N_NODES - FLUSH_R)
        pltpu.sync_copy(z_hbm, rp_acc.at[pl.ds(fb, FLUSH_R)])
        plsc.subcore_barrier()

        pltpu.async_copy(wmeta_hbm.at[s, 0], wbuf_v.at[0], sem_w0)
        pltpu.async_copy(wmeta_hbm.at[s, 1], wbuf_v.at[1], sem_w1)
        unpack(0, 0, qbase)
        pltpu.async_copy(h_hbm.at[sidx_v.at[0]], rows_v.at[0], sem_g0)
        unpack(1, 1, qbase)
        pltpu.async_copy(h_hbm.at[sidx_v.at[1]], rows_v.at[1], sem_g1)

        def super_chunk(S, carry):
            for b in range(4):
                rb = b % 2
                b2 = (b + 2) % 4
                i = 4 * S + b
                pltpu.make_async_copy(dummy_g, rows_v.at[b], sem_g[b]).wait()

                @pl.when(i >= 2)
                def _drain_s():
                    pltpu.make_async_copy(dummy_s, stg_v.at[rb, 0], sem_s[rb][0]).wait()
                    pltpu.make_async_copy(dummy_s, stg_v.at[rb, 1], sem_s[rb][1]).wait()

                pltpu.make_async_copy(dummy_w, wbuf_v.at[b], sem_w[b]).wait()

                @pl.when(i + 2 < NCHUNK_E)
                def _prefetch():
                    unpack(i + 2, b2, qbase)
                    pltpu.async_copy(h_hbm.at[sidx_v.at[b2]], rows_v.at[b2], sem_g[b2])
                    pltpu.async_copy(wmeta_hbm.at[s, i + 2], wbuf_v.at[b2], sem_w[b2])

                def group(g, cc):
                    ws16 = wbuf_v[b, 0, pl.ds(g * 16, 16)]
                    wp16 = wbuf_v[b, 1, pl.ds(g * 16, 16)]
                    for j2 in range(16):
                        wsv = jnp.full((16,), ws16[j2], dtype=jnp.float32)
                        wpv = jnp.full((16,), wp16[j2], dtype=jnp.float32)
                        jj = g * 16 + j2
                        gh = g // 2
                        jh = (g % 2) * 16 + j2
                        for f in range(FQ // 16):
                            r = rows_v[b, jj, pl.ds(f * 16, 16)]
                            stg_v[rb, gh, jh, pl.ds(f * 16, 16)] = r * wsv
                            stg_v[rb, gh, jh, pl.ds(FQ + f * 16, 16)] = r * wpv
                    return cc

                lax.fori_loop(0, CE // 16, group, 0)
                pltpu.async_copy(stg_v.at[rb, 0], rp_acc.at[didx_v.at[b, 0]],
                                 sem_s[rb][0], add=True)
                pltpu.async_copy(stg_v.at[rb, 1], rp_acc.at[didx_v.at[b, 1]],
                                 sem_s[rb][1], add=True)
            return carry

        lax.fori_loop(0, NCHUNK_E // 4, super_chunk, 0)
        pltpu.make_async_copy(dummy_s, stg_v.at[0, 0], sem_s0a).wait()
        pltpu.make_async_copy(dummy_s, stg_v.at[0, 1], sem_s0b).wait()
        pltpu.make_async_copy(dummy_s, stg_v.at[1, 0], sem_s1a).wait()
        pltpu.make_async_copy(dummy_s, stg_v.at[1, 1], sem_s1b).wait()
        plsc.subcore_barrier()
        pltpu.sync_copy(rp_acc.at[pl.ds(fb, FLUSH_R)],
                        rp_hbm.at[pl.ds(qbase + fb, FLUSH_R)])
        plsc.subcore_barrier()


def _edge_stage(hflat, meta_sd, wmeta, zrows):
    mesh = plsc.VectorSubcoreMesh(core_axis_name="c", subcore_axis_name="s")
    return pl.kernel(
        _edge_body,
        out_type=jax.ShapeDtypeStruct((Q * N_NODES, 2 * FQ), jnp.float32),
        mesh=mesh,
        compiler_params=pltpu.CompilerParams(use_tc_tiling_on_sc=False),
        scratch_types=[
            pltpu.VMEM((NCHUNK_E, CE), jnp.int32),
            pltpu.VMEM((4, CE), jnp.int32),
            pltpu.VMEM((4, 2, CE // 2), jnp.int32),
            pltpu.VMEM((4, 2, CE), jnp.float32),
            pltpu.VMEM((4, CE, FQ), jnp.float32),
            pltpu.VMEM((2, 2, CE // 2, 2 * FQ), jnp.float32),
            pltpu.VMEM_SHARED((N_NODES, 2 * FQ), jnp.float32),
        ] + [pltpu.SemaphoreType.DMA] * 12,
    )(hflat, meta_sd, wmeta, zrows)


_RB = 1000


def _layer_body_split(rp_ref, wt_ref, sg_ref, bb_ref, out_ref):
    acc = jnp.dot(rp_ref[0][:, FQ:], wt_ref[0], preferred_element_type=jnp.float32)
    for q in range(1, Q):
        acc = acc + jnp.dot(rp_ref[q][:, FQ:], wt_ref[q], preferred_element_type=jnp.float32)
    z = jnp.maximum(acc * sg_ref[...] + bb_ref[...], 0.0)
    for q in range(Q):
        out_ref[q] = z[:, q * FQ:(q + 1) * FQ] + rp_ref[q][:, :FQ]


def _layer_body_final(rp_ref, wt_ref, sg_ref, bb_ref, out_ref):
    acc = jnp.dot(rp_ref[0][:, FQ:], wt_ref[0], preferred_element_type=jnp.float32)
    for q in range(1, Q):
        acc = acc + jnp.dot(rp_ref[q][:, FQ:], wt_ref[q], preferred_element_type=jnp.float32)
    z = jnp.maximum(acc * sg_ref[...] + bb_ref[...], 0.0)
    cat = jnp.concatenate([rp_ref[q][:, :FQ] for q in range(Q)], axis=1)
    out_ref[...] = z + cat


def _layer_stage(rp4, W, b, g, bb, final):
    sg = (g * _BN_S)[None, :]
    bb2 = (bb + b * _BN_S * g)[None, :]
    wt4 = W.T.reshape(Q, FQ, HIDDEN)
    grid = (N_NODES // _RB,)
    rp_spec = pl.BlockSpec((Q, _RB, 2 * FQ), lambda i: (0, i, 0))
    if final:
        body, out_shape = _layer_body_final, jax.ShapeDtypeStruct((N_NODES, HIDDEN), jnp.float32)
        out_spec = pl.BlockSpec((_RB, HIDDEN), lambda i: (i, 0))
    else:
        body, out_shape = _layer_body_split, jax.ShapeDtypeStruct((Q, N_NODES, FQ), jnp.float32)
        out_spec = pl.BlockSpec((Q, _RB, FQ), lambda i: (0, i, 0))
    return pl.pallas_call(
        body,
        out_shape=out_shape,
        grid=grid,
        in_specs=[
            rp_spec,
            pl.BlockSpec((Q, FQ, HIDDEN), lambda i: (0, 0, 0)),
            pl.BlockSpec((1, HIDDEN), lambda i: (0, 0)),
            pl.BlockSpec((1, HIDDEN), lambda i: (0, 0)),
        ],
        out_specs=out_spec,
    )(rp4, wt4, sg, bb2)


def kernel(input_ids, offsets, edge_index, self_w, ppi_w, emb_table, input_bias,
           bn0_g, bn0_b, W1, b1, bn1_g, bn1_b, W2, b2, bn2_g, bn2_b):
    sg0 = bn0_g * _BN_S
    bb0 = bn0_b + input_bias * _BN_S * bn0_g
    hflat = _emb_stage(input_ids, emb_table, sg0, bb0)
    pad = NS * EPT_PAD - N_EDGES
    srcp = jnp.concatenate([edge_index[0], jnp.zeros((pad,), jnp.int32)])
    dstp = jnp.concatenate([edge_index[1], jnp.zeros((pad,), jnp.int32)])
    swp = jnp.concatenate([self_w, jnp.zeros((pad,), jnp.float32)])
    pwp = jnp.concatenate([ppi_w, jnp.zeros((pad,), jnp.float32)])
    meta_sd = (srcp | (dstp << 16)).reshape(NS, NCHUNK_E, CE)
    wmeta = jnp.stack([swp.reshape(NS, NCHUNK_E, CE),
                       pwp.reshape(NS, NCHUNK_E, CE)], axis=2)
    zrows = jnp.zeros((FLUSH_R, 2 * FQ), jnp.float32)
    for (W, b, g, bb, final) in ((W1, b1, bn1_g, bn1_b, False),
                                 (W2, b2, bn2_g, bn2_b, True)):
        rpflat = _edge_stage(hflat, meta_sd, wmeta, zrows)
        out = _layer_stage(rpflat.reshape(Q, N_NODES, 2 * FQ), W, b, g, bb, final)
        if not final:
            hflat = out.reshape(Q * N_NODES, FQ)
    return out


# speedup vs baseline: 11.3620x; 1.3044x over previous
import functools
import math

import jax
import jax.numpy as jnp
from jax import lax
from jax.experimental import pallas as pl
from jax.experimental.pallas import tpu as pltpu
from jax.experimental.pallas import tpu_sc as plsc

N_NODES = 10000
HIDDEN = 256
BAG = 10
N_EDGES = 160000
_BN_S = 1.0 / math.sqrt(1.0 + 1e-5)

NC = 2
NS = 16
NW = NC * NS
Q = 4
FQ = HIDDEN // Q

BPW = 320
CB = 16
NCHUNK_B = BPW // CB


def _emb_body(ids_hbm, table_hbm, sg_hbm, bb_hbm, out_hbm,
              idx_a, idx_b, rows_v, sg_v, bb_v, oq_v, sem):
    c = lax.axis_index("c")
    s = lax.axis_index("s")
    w = c * NS + s
    pltpu.sync_copy(sg_hbm, sg_v)
    pltpu.sync_copy(bb_hbm, bb_v)

    def chunk(i, carry):
        start = jnp.minimum(w * BPW + i * CB, N_NODES - CB)
        half = CB * BAG // 2
        pltpu.sync_copy(ids_hbm.at[pl.ds(start * BAG, half)], idx_a)
        pltpu.sync_copy(ids_hbm.at[pl.ds(start * BAG + half, half)], idx_b)
        cp_a = pltpu.async_copy(table_hbm.at[idx_a], rows_v.at[pl.ds(0, half)], sem)
        cp_b = pltpu.async_copy(table_hbm.at[idx_b], rows_v.at[pl.ds(half, half)], sem)
        cp_a.wait()
        cp_b.wait()

        def bag(b, carry2):
            base = b * BAG
            for f in range(HIDDEN // 16):
                acc = rows_v[base, pl.ds(f * 16, 16)]
                for k in range(1, BAG):
                    acc = acc + rows_v[base + k, pl.ds(f * 16, 16)]
                val = acc * sg_v[pl.ds(f * 16, 16)] + bb_v[pl.ds(f * 16, 16)]
                val = jnp.maximum(val, 0.0)
                q, fq = f // (FQ // 16), f % (FQ // 16)
                oq_v[q, b, pl.ds(fq * 16, 16)] = val
            return carry2

        lax.fori_loop(0, CB, bag, 0)
        for q in range(Q):
            pltpu.sync_copy(oq_v.at[q], out_hbm.at[pl.ds(q * N_NODES + start, CB)])
        return carry

    lax.fori_loop(0, NCHUNK_B, chunk, 0)


def _emb_stage(input_ids, emb_table, sg, bb):
    mesh = plsc.VectorSubcoreMesh(core_axis_name="c", subcore_axis_name="s")
    return pl.kernel(
        _emb_body,
        out_type=jax.ShapeDtypeStruct((Q * N_NODES, FQ), jnp.float32),
        mesh=mesh,
        scratch_types=[
            pltpu.VMEM((CB * BAG // 2,), jnp.int32),
            pltpu.VMEM((CB * BAG // 2,), jnp.int32),
            pltpu.VMEM((CB * BAG, HIDDEN), jnp.float32),
            pltpu.VMEM((HIDDEN,), jnp.float32),
            pltpu.VMEM((HIDDEN,), jnp.float32),
            pltpu.VMEM((Q, CB, FQ), jnp.float32),
            pltpu.SemaphoreType.DMA,
        ],
    )(input_ids, emb_table, sg, bb)


CE = 64
NCHUNK_E = 160
EPT_PAD = NCHUNK_E * CE
FLUSH_R = 640


def _edge_body(h_hbm, meta_hbm, wmeta_hbm, z_hbm, rp_hbm,
               meta_v, sidx_v, didx_v, wbuf_v, rows_v, stg_v, rp_acc,
               sem_g0, sem_g1, sem_g2, sem_g3,
               sem_w0, sem_w1, sem_w2, sem_w3,
               sem_s0a, sem_s0b, sem_s1a, sem_s1b):
    c = lax.axis_index("c")
    s = lax.axis_index("s")
    pltpu.sync_copy(meta_hbm.at[s], meta_v)
    dummy_g = h_hbm.at[pl.ds(0, CE)]
    dummy_s = rp_hbm.at[pl.ds(0, CE // 2)]
    dummy_w = wmeta_hbm.at[0, 0]
    sem_g = (sem_g0, sem_g1, sem_g2, sem_g3)
    sem_w = (sem_w0, sem_w1, sem_w2, sem_w3)
    sem_s = ((sem_s0a, sem_s0b), (sem_s1a, sem_s1b))

    def unpack(i, b4, qbase):
        for t in range(CE // 16):
            m = meta_v[i, pl.ds(t * 16, 16)]
            sidx_v[b4, pl.ds(t * 16, 16)] = (m & 0xFFFF) + qbase
            didx_v[b4, t // 2, pl.ds((t % 2) * 16, 16)] = lax.shift_right_logical(m, 16)

    for p in range(2):
        q = c * 2 + p
        qbase = q * N_NODES
        fb = jnp.minimum(s * FLUSH_R, N_NODES - FLUSH_R)
        pltpu.sync_copy(z_hbm, rp_acc.at[pl.ds(fb, FLUSH_R)])
        plsc.subcore_barrier()

        pltpu.async_copy(wmeta_hbm.at[s, 0], wbuf_v.at[0], sem_w0)
        pltpu.async_copy(wmeta_hbm.at[s, 1], wbuf_v.at[1], sem_w1)
        unpack(0, 0, qbase)
        pltpu.async_copy(h_hbm.at[sidx_v.at[0]], rows_v.at[0], sem_g0)
        unpack(1, 1, qbase)
        pltpu.async_copy(h_hbm.at[sidx_v.at[1]], rows_v.at[1], sem_g1)

        def super_chunk(S, carry):
            for b in range(4):
                rb = b % 2
                b2 = (b + 2) % 4
                i = 4 * S + b
                pltpu.make_async_copy(dummy_g, rows_v.at[b], sem_g[b]).wait()

                @pl.when(i >= 2)
                def _drain_s():
                    pltpu.make_async_copy(dummy_s, stg_v.at[rb, 0], sem_s[rb][0]).wait()
                    pltpu.make_async_copy(dummy_s, stg_v.at[rb, 1], sem_s[rb][1]).wait()

                pltpu.make_async_copy(dummy_w, wbuf_v.at[b], sem_w[b]).wait()

                @pl.when(i + 2 < NCHUNK_E)
                def _prefetch():
                    unpack(i + 2, b2, qbase)
                    pltpu.async_copy(h_hbm.at[sidx_v.at[b2]], rows_v.at[b2], sem_g[b2])
                    pltpu.async_copy(wmeta_hbm.at[s, i + 2], wbuf_v.at[b2], sem_w[b2])

                for g in range(CE // 16):
                    ws16 = wbuf_v[b, 0, pl.ds(g * 16, 16)]
                    wp16 = wbuf_v[b, 1, pl.ds(g * 16, 16)]
                    for j2 in range(16):
                        wsv = jnp.full((16,), ws16[j2], dtype=jnp.float32)
                        wpv = jnp.full((16,), wp16[j2], dtype=jnp.float32)
                        jj = g * 16 + j2
                        gh = g // 2
                        jh = (g % 2) * 16 + j2
                        for f in range(FQ // 16):
                            r = rows_v[b, jj, pl.ds(f * 16, 16)]
                            stg_v[rb, gh, jh, pl.ds(f * 16, 16)] = r * wsv
                            stg_v[rb, gh, jh, pl.ds(FQ + f * 16, 16)] = r * wpv
                pltpu.async_copy(stg_v.at[rb, 0], rp_acc.at[didx_v.at[b, 0]],
                                 sem_s[rb][0], add=True)
                pltpu.async_copy(stg_v.at[rb, 1], rp_acc.at[didx_v.at[b, 1]],
                                 sem_s[rb][1], add=True)
            return carry

        lax.fori_loop(0, NCHUNK_E // 4, super_chunk, 0)
        pltpu.make_async_copy(dummy_s, stg_v.at[0, 0], sem_s0a).wait()
        pltpu.make_async_copy(dummy_s, stg_v.at[0, 1], sem_s0b).wait()
        pltpu.make_async_copy(dummy_s, stg_v.at[1, 0], sem_s1a).wait()
        pltpu.make_async_copy(dummy_s, stg_v.at[1, 1], sem_s1b).wait()
        plsc.subcore_barrier()
        pltpu.sync_copy(rp_acc.at[pl.ds(fb, FLUSH_R)],
                        rp_hbm.at[pl.ds(qbase + fb, FLUSH_R)])
        plsc.subcore_barrier()


def _edge_stage(hflat, meta_sd, wmeta, zrows):
    mesh = plsc.VectorSubcoreMesh(core_axis_name="c", subcore_axis_name="s")
    return pl.kernel(
        _edge_body,
        out_type=jax.ShapeDtypeStruct((Q * N_NODES, 2 * FQ), jnp.float32),
        mesh=mesh,
        compiler_params=pltpu.CompilerParams(use_tc_tiling_on_sc=False),
        scratch_types=[
            pltpu.VMEM((NCHUNK_E, CE), jnp.int32),
            pltpu.VMEM((4, CE), jnp.int32),
            pltpu.VMEM((4, 2, CE // 2), jnp.int32),
            pltpu.VMEM((4, 2, CE), jnp.float32),
            pltpu.VMEM((4, CE, FQ), jnp.float32),
            pltpu.VMEM((2, 2, CE // 2, 2 * FQ), jnp.float32),
            pltpu.VMEM_SHARED((N_NODES, 2 * FQ), jnp.float32),
        ] + [pltpu.SemaphoreType.DMA] * 12,
    )(hflat, meta_sd, wmeta, zrows)


_RB = 1000


def _layer_body_split(rp_ref, wt_ref, sg_ref, bb_ref, out_ref):
    acc = jnp.dot(rp_ref[0][:, FQ:], wt_ref[0], preferred_element_type=jnp.float32)
    for q in range(1, Q):
        acc = acc + jnp.dot(rp_ref[q][:, FQ:], wt_ref[q], preferred_element_type=jnp.float32)
    z = jnp.maximum(acc * sg_ref[...] + bb_ref[...], 0.0)
    for q in range(Q):
        out_ref[q] = z[:, q * FQ:(q + 1) * FQ] + rp_ref[q][:, :FQ]


def _layer_body_final(rp_ref, wt_ref, sg_ref, bb_ref, out_ref):
    acc = jnp.dot(rp_ref[0][:, FQ:], wt_ref[0], preferred_element_type=jnp.float32)
    for q in range(1, Q):
        acc = acc + jnp.dot(rp_ref[q][:, FQ:], wt_ref[q], preferred_element_type=jnp.float32)
    z = jnp.maximum(acc * sg_ref[...] + bb_ref[...], 0.0)
    cat = jnp.concatenate([rp_ref[q][:, :FQ] for q in range(Q)], axis=1)
    out_ref[...] = z + cat


def _layer_stage(rp4, W, b, g, bb, final):
    sg = (g * _BN_S)[None, :]
    bb2 = (bb + b * _BN_S * g)[None, :]
    wt4 = W.T.reshape(Q, FQ, HIDDEN)
    grid = (N_NODES // _RB,)
    rp_spec = pl.BlockSpec((Q, _RB, 2 * FQ), lambda i: (0, i, 0))
    if final:
        body, out_shape = _layer_body_final, jax.ShapeDtypeStruct((N_NODES, HIDDEN), jnp.float32)
        out_spec = pl.BlockSpec((_RB, HIDDEN), lambda i: (i, 0))
    else:
        body, out_shape = _layer_body_split, jax.ShapeDtypeStruct((Q, N_NODES, FQ), jnp.float32)
        out_spec = pl.BlockSpec((Q, _RB, FQ), lambda i: (0, i, 0))
    return pl.pallas_call(
        body,
        out_shape=out_shape,
        grid=grid,
        in_specs=[
            rp_spec,
            pl.BlockSpec((Q, FQ, HIDDEN), lambda i: (0, 0, 0)),
            pl.BlockSpec((1, HIDDEN), lambda i: (0, 0)),
            pl.BlockSpec((1, HIDDEN), lambda i: (0, 0)),
        ],
        out_specs=out_spec,
    )(rp4, wt4, sg, bb2)


def kernel(input_ids, offsets, edge_index, self_w, ppi_w, emb_table, input_bias,
           bn0_g, bn0_b, W1, b1, bn1_g, bn1_b, W2, b2, bn2_g, bn2_b):
    sg0 = bn0_g * _BN_S
    bb0 = bn0_b + input_bias * _BN_S * bn0_g
    hflat = _emb_stage(input_ids, emb_table, sg0, bb0)
    pad = NS * EPT_PAD - N_EDGES
    srcp = jnp.concatenate([edge_index[0], jnp.zeros((pad,), jnp.int32)])
    dstp = jnp.concatenate([edge_index[1], jnp.zeros((pad,), jnp.int32)])
    swp = jnp.concatenate([self_w, jnp.zeros((pad,), jnp.float32)])
    pwp = jnp.concatenate([ppi_w, jnp.zeros((pad,), jnp.float32)])
    meta_sd = (srcp | (dstp << 16)).reshape(NS, NCHUNK_E, CE)
    wmeta = jnp.stack([swp.reshape(NS, NCHUNK_E, CE),
                       pwp.reshape(NS, NCHUNK_E, CE)], axis=2)
    zrows = jnp.zeros((FLUSH_R, 2 * FQ), jnp.float32)
    for (W, b, g, bb, final) in ((W1, b1, bn1_g, bn1_b, False),
                                 (W2, b2, bn2_g, bn2_b, True)):
        rpflat = _edge_stage(hflat, meta_sd, wmeta, zrows)
        out = _layer_stage(rpflat.reshape(Q, N_NODES, 2 * FQ), W, b, g, bb, final)
        if not final:
            hflat = out.reshape(Q * N_NODES, FQ)
    return out


# speedup vs baseline: 11.8848x; 1.0460x over previous
import functools
import math

import jax
import jax.numpy as jnp
from jax import lax
from jax.experimental import pallas as pl
from jax.experimental.pallas import tpu as pltpu
from jax.experimental.pallas import tpu_sc as plsc

N_NODES = 10000
HIDDEN = 256
BAG = 10
N_EDGES = 160000
_BN_S = 1.0 / math.sqrt(1.0 + 1e-5)

NC = 2
NS = 16
NW = NC * NS
Q = 4
FQ = HIDDEN // Q

BPW = 320
CB = 16
NCHUNK_B = BPW // CB


def _emb_body(ids_hbm, table_hbm, sg_hbm, bb_hbm, out_hbm,
              idx_v, rows_v, sg_v, bb_v, oq_v,
              sem_ga0, sem_gb0, sem_ga1, sem_gb1, sem_o0, sem_o1):
    c = lax.axis_index("c")
    s = lax.axis_index("s")
    w = c * NS + s
    pltpu.sync_copy(sg_hbm, sg_v)
    pltpu.sync_copy(bb_hbm, bb_v)
    half = CB * BAG // 2
    sem_ga = (sem_ga0, sem_ga1)
    sem_gb = (sem_gb0, sem_gb1)
    sem_o = (sem_o0, sem_o1)
    dummy_r = table_hbm.at[pl.ds(0, half)]
    dummy_o = out_hbm.at[pl.ds(0, CB)]

    def chunk_start(i):
        return jnp.minimum(w * BPW + i * CB, N_NODES - CB)

    def fetch(i, b):
        st = chunk_start(i)
        pltpu.sync_copy(ids_hbm.at[pl.ds(st * BAG, half)], idx_v.at[b, 0])
        pltpu.sync_copy(ids_hbm.at[pl.ds(st * BAG + half, half)], idx_v.at[b, 1])
        pltpu.async_copy(table_hbm.at[idx_v.at[b, 0]], rows_v.at[b, pl.ds(0, half)],
                         sem_ga[b])
        pltpu.async_copy(table_hbm.at[idx_v.at[b, 1]], rows_v.at[b, pl.ds(half, half)],
                         sem_gb[b])

    fetch(0, 0)
    fetch(1, 1)

    def super_chunk(S, carry):
        for b in range(2):
            i = 2 * S + b
            pltpu.make_async_copy(dummy_r, rows_v.at[b, pl.ds(0, half)], sem_ga[b]).wait()
            pltpu.make_async_copy(dummy_r, rows_v.at[b, pl.ds(half, half)], sem_gb[b]).wait()

            @pl.when(i >= 2)
            def _drain_o():
                for qq in range(Q):
                    pltpu.make_async_copy(dummy_o, oq_v.at[b, 0], sem_o[b]).wait()

            def bagpair(k, cc):
                for u in range(2):
                    base = (k * 2 + u) * BAG
                    for f in range(HIDDEN // 16):
                        acc = rows_v[b, base, pl.ds(f * 16, 16)]
                        for kk in range(1, BAG):
                            acc = acc + rows_v[b, base + kk, pl.ds(f * 16, 16)]
                        val = acc * sg_v[pl.ds(f * 16, 16)] + bb_v[pl.ds(f * 16, 16)]
                        val = jnp.maximum(val, 0.0)
                        qq, fq = f // (FQ // 16), f % (FQ // 16)
                        oq_v[b, qq, k * 2 + u, pl.ds(fq * 16, 16)] = val
                return cc

            lax.fori_loop(0, CB // 2, bagpair, 0)
            st = chunk_start(i)
            for qq in range(Q):
                pltpu.async_copy(oq_v.at[b, qq],
                                 out_hbm.at[pl.ds(qq * N_NODES + st, CB)], sem_o[b])

            @pl.when(i + 2 < NCHUNK_B)
            def _prefetch():
                fetch(i + 2, b)
        return carry

    lax.fori_loop(0, NCHUNK_B // 2, super_chunk, 0)
    for b in range(2):
        for qq in range(Q):
            pltpu.make_async_copy(dummy_o, oq_v.at[b, 0], sem_o[b]).wait()


def _emb_stage(input_ids, emb_table, sg, bb):
    mesh = plsc.VectorSubcoreMesh(core_axis_name="c", subcore_axis_name="s")
    return pl.kernel(
        _emb_body,
        out_type=jax.ShapeDtypeStruct((Q * N_NODES, FQ), jnp.float32),
        mesh=mesh,
        scratch_types=[
            pltpu.VMEM((2, 2, CB * BAG // 2), jnp.int32),
            pltpu.VMEM((2, CB * BAG, HIDDEN), jnp.float32),
            pltpu.VMEM((HIDDEN,), jnp.float32),
            pltpu.VMEM((HIDDEN,), jnp.float32),
            pltpu.VMEM((2, Q, CB, FQ), jnp.float32),
        ] + [pltpu.SemaphoreType.DMA] * 6,
    )(input_ids, emb_table, sg, bb)


CE = 64
NCHUNK_E = 160
EPT_PAD = NCHUNK_E * CE
FLUSH_R = 640


def _edge_body(h_hbm, meta_hbm, wmeta_hbm, z_hbm, rp_hbm,
               meta_v, sidx_v, didx_v, wbuf_v, rows_v, stg_v, rp_acc,
               sem_g0, sem_g1, sem_g2, sem_g3,
               sem_w0, sem_w1, sem_w2, sem_w3,
               sem_s0a, sem_s0b, sem_s1a, sem_s1b):
    c = lax.axis_index("c")
    s = lax.axis_index("s")
    pltpu.sync_copy(meta_hbm.at[s], meta_v)
    dummy_g = h_hbm.at[pl.ds(0, CE)]
    dummy_s = rp_hbm.at[pl.ds(0, CE // 2)]
    dummy_w = wmeta_hbm.at[0, 0]
    sem_g = (sem_g0, sem_g1, sem_g2, sem_g3)
    sem_w = (sem_w0, sem_w1, sem_w2, sem_w3)
    sem_s = ((sem_s0a, sem_s0b), (sem_s1a, sem_s1b))

    def unpack(i, b4, qbase):
        for t in range(CE // 16):
            m = meta_v[i, pl.ds(t * 16, 16)]
            sidx_v[b4, pl.ds(t * 16, 16)] = (m & 0xFFFF) + qbase
            didx_v[b4, t // 2, pl.ds((t % 2) * 16, 16)] = lax.shift_right_logical(m, 16)

    for p in range(2):
        q = c * 2 + p
        qbase = q * N_NODES
        fb = jnp.minimum(s * FLUSH_R, N_NODES - FLUSH_R)
        pltpu.sync_copy(z_hbm, rp_acc.at[pl.ds(fb, FLUSH_R)])
        plsc.subcore_barrier()

        pltpu.async_copy(wmeta_hbm.at[s, 0], wbuf_v.at[0], sem_w0)
        pltpu.async_copy(wmeta_hbm.at[s, 1], wbuf_v.at[1], sem_w1)
        unpack(0, 0, qbase)
        pltpu.async_copy(h_hbm.at[sidx_v.at[0]], rows_v.at[0], sem_g0)
        unpack(1, 1, qbase)
        pltpu.async_copy(h_hbm.at[sidx_v.at[1]], rows_v.at[1], sem_g1)

        def super_chunk(S, carry):
            for b in range(4):
                rb = b % 2
                b2 = (b + 2) % 4
                i = 4 * S + b
                pltpu.make_async_copy(dummy_g, rows_v.at[b], sem_g[b]).wait()

                @pl.when(i >= 2)
                def _drain_s():
                    pltpu.make_async_copy(dummy_s, stg_v.at[rb, 0], sem_s[rb][0]).wait()
                    pltpu.make_async_copy(dummy_s, stg_v.at[rb, 1], sem_s[rb][1]).wait()

                pltpu.make_async_copy(dummy_w, wbuf_v.at[b], sem_w[b]).wait()

                @pl.when(i + 2 < NCHUNK_E)
                def _prefetch():
                    unpack(i + 2, b2, qbase)
                    pltpu.async_copy(h_hbm.at[sidx_v.at[b2]], rows_v.at[b2], sem_g[b2])
                    pltpu.async_copy(wmeta_hbm.at[s, i + 2], wbuf_v.at[b2], sem_w[b2])

                for g in range(CE // 16):
                    ws16 = wbuf_v[b, 0, pl.ds(g * 16, 16)]
                    wp16 = wbuf_v[b, 1, pl.ds(g * 16, 16)]
                    for j2 in range(16):
                        wsv = jnp.full((16,), ws16[j2], dtype=jnp.float32)
                        wpv = jnp.full((16,), wp16[j2], dtype=jnp.float32)
                        jj = g * 16 + j2
                        gh = g // 2
                        jh = (g % 2) * 16 + j2
                        for f in range(FQ // 16):
                            r = rows_v[b, jj, pl.ds(f * 16, 16)]
                            stg_v[rb, gh, jh, pl.ds(f * 16, 16)] = r * wsv
                            stg_v[rb, gh, jh, pl.ds(FQ + f * 16, 16)] = r * wpv
                pltpu.async_copy(stg_v.at[rb, 0], rp_acc.at[didx_v.at[b, 0]],
                                 sem_s[rb][0], add=True)
                pltpu.async_copy(stg_v.at[rb, 1], rp_acc.at[didx_v.at[b, 1]],
                                 sem_s[rb][1], add=True)
            return carry

        lax.fori_loop(0, NCHUNK_E // 4, super_chunk, 0)
        pltpu.make_async_copy(dummy_s, stg_v.at[0, 0], sem_s0a).wait()
        pltpu.make_async_copy(dummy_s, stg_v.at[0, 1], sem_s0b).wait()
        pltpu.make_async_copy(dummy_s, stg_v.at[1, 0], sem_s1a).wait()
        pltpu.make_async_copy(dummy_s, stg_v.at[1, 1], sem_s1b).wait()
        plsc.subcore_barrier()
        pltpu.sync_copy(rp_acc.at[pl.ds(fb, FLUSH_R)],
                        rp_hbm.at[pl.ds(qbase + fb, FLUSH_R)])
        plsc.subcore_barrier()


def _edge_stage(hflat, meta_sd, wmeta, zrows):
    mesh = plsc.VectorSubcoreMesh(core_axis_name="c", subcore_axis_name="s")
    return pl.kernel(
        _edge_body,
        out_type=jax.ShapeDtypeStruct((Q * N_NODES, 2 * FQ), jnp.float32),
        mesh=mesh,
        compiler_params=pltpu.CompilerParams(use_tc_tiling_on_sc=False),
        scratch_types=[
            pltpu.VMEM((NCHUNK_E, CE), jnp.int32),
            pltpu.VMEM((4, CE), jnp.int32),
            pltpu.VMEM((4, 2, CE // 2), jnp.int32),
            pltpu.VMEM((4, 2, CE), jnp.float32),
            pltpu.VMEM((4, CE, FQ), jnp.float32),
            pltpu.VMEM((2, 2, CE // 2, 2 * FQ), jnp.float32),
            pltpu.VMEM_SHARED((N_NODES, 2 * FQ), jnp.float32),
        ] + [pltpu.SemaphoreType.DMA] * 12,
    )(hflat, meta_sd, wmeta, zrows)


_RB = 1000


def _layer_body_split(rp_ref, wt_ref, sg_ref, bb_ref, out_ref):
    acc = jnp.dot(rp_ref[0][:, FQ:], wt_ref[0], preferred_element_type=jnp.float32)
    for q in range(1, Q):
        acc = acc + jnp.dot(rp_ref[q][:, FQ:], wt_ref[q], preferred_element_type=jnp.float32)
    z = jnp.maximum(acc * sg_ref[...] + bb_ref[...], 0.0)
    for q in range(Q):
        out_ref[q] = z[:, q * FQ:(q + 1) * FQ] + rp_ref[q][:, :FQ]


def _layer_body_final(rp_ref, wt_ref, sg_ref, bb_ref, out_ref):
    acc = jnp.dot(rp_ref[0][:, FQ:], wt_ref[0], preferred_element_type=jnp.float32)
    for q in range(1, Q):
        acc = acc + jnp.dot(rp_ref[q][:, FQ:], wt_ref[q], preferred_element_type=jnp.float32)
    z = jnp.maximum(acc * sg_ref[...] + bb_ref[...], 0.0)
    cat = jnp.concatenate([rp_ref[q][:, :FQ] for q in range(Q)], axis=1)
    out_ref[...] = z + cat


def _layer_stage(rp4, W, b, g, bb, final):
    sg = (g * _BN_S)[None, :]
    bb2 = (bb + b * _BN_S * g)[None, :]
    wt4 = W.T.reshape(Q, FQ, HIDDEN)
    grid = (N_NODES // _RB,)
    rp_spec = pl.BlockSpec((Q, _RB, 2 * FQ), lambda i: (0, i, 0))
    if final:
        body, out_shape = _layer_body_final, jax.ShapeDtypeStruct((N_NODES, HIDDEN), jnp.float32)
        out_spec = pl.BlockSpec((_RB, HIDDEN), lambda i: (i, 0))
    else:
        body, out_shape = _layer_body_split, jax.ShapeDtypeStruct((Q, N_NODES, FQ), jnp.float32)
        out_spec = pl.BlockSpec((Q, _RB, FQ), lambda i: (0, i, 0))
    return pl.pallas_call(
        body,
        out_shape=out_shape,
        grid=grid,
        in_specs=[
            rp_spec,
            pl.BlockSpec((Q, FQ, HIDDEN), lambda i: (0, 0, 0)),
            pl.BlockSpec((1, HIDDEN), lambda i: (0, 0)),
            pl.BlockSpec((1, HIDDEN), lambda i: (0, 0)),
        ],
        out_specs=out_spec,
    )(rp4, wt4, sg, bb2)


def kernel(input_ids, offsets, edge_index, self_w, ppi_w, emb_table, input_bias,
           bn0_g, bn0_b, W1, b1, bn1_g, bn1_b, W2, b2, bn2_g, bn2_b):
    sg0 = bn0_g * _BN_S
    bb0 = bn0_b + input_bias * _BN_S * bn0_g
    hflat = _emb_stage(input_ids, emb_table, sg0, bb0)
    pad = NS * EPT_PAD - N_EDGES
    srcp = jnp.concatenate([edge_index[0], jnp.zeros((pad,), jnp.int32)])
    dstp = jnp.concatenate([edge_index[1], jnp.zeros((pad,), jnp.int32)])
    swp = jnp.concatenate([self_w, jnp.zeros((pad,), jnp.float32)])
    pwp = jnp.concatenate([ppi_w, jnp.zeros((pad,), jnp.float32)])
    meta_sd = (srcp | (dstp << 16)).reshape(NS, NCHUNK_E, CE)
    wmeta = jnp.stack([swp.reshape(NS, NCHUNK_E, CE),
                       pwp.reshape(NS, NCHUNK_E, CE)], axis=2)
    zrows = jnp.zeros((FLUSH_R, 2 * FQ), jnp.float32)
    for (W, b, g, bb, final) in ((W1, b1, bn1_g, bn1_b, False),
                                 (W2, b2, bn2_g, bn2_b, True)):
        rpflat = _edge_stage(hflat, meta_sd, wmeta, zrows)
        out = _layer_stage(rpflat.reshape(Q, N_NODES, 2 * FQ), W, b, g, bb, final)
        if not final:
            hflat = out.reshape(Q * N_NODES, FQ)
    return out
